# baseline (speedup 1.0000x reference)
import functools

import jax
import jax.numpy as jnp
from jax import lax
from jax.experimental import pallas as pl
from jax.experimental.pallas import tpu as pltpu
from jax.experimental.pallas import tpu_sc as plsc

B = 32768
K = 16
D = 128
NSEG = 2
BSEG = B // NSEG
NC = 2
NS = 16
NW = NC * NS
ITEMS = BSEG // NW
C = 64
NCHUNK = ITEMS // C
SUB = 8
NSUB = C // SUB
LANES = 16
DV = D // LANES


def _sc_gather_weighted(node_ids, table, nbr_ids, nbr_w):
  mesh = plsc.VectorSubcoreMesh(core_axis_name="c", subcore_axis_name="s")

  @functools.partial(
      pl.kernel,
      out_type=[
          jax.ShapeDtypeStruct((BSEG, D), jnp.float32),
          jax.ShapeDtypeStruct((BSEG, D), jnp.float32),
      ],
      mesh=mesh,
      compiler_params=pltpu.CompilerParams(use_tc_tiling_on_sc=False),
      scratch_types=[
          pltpu.VMEM((C,), jnp.int32),
          pltpu.VMEM((C, K), jnp.int32),
          pltpu.VMEM((C * K,), jnp.int32),
          pltpu.VMEM((C, K), jnp.float32),
          pltpu.VMEM((C, D), jnp.float32),
          pltpu.VMEM((SUB * K, D), jnp.float32),
          pltpu.VMEM((SUB * K, D), jnp.float32),
          pltpu.VMEM((SUB * K, D), jnp.float32),
          pltpu.VMEM((SUB * K, D), jnp.float32),
          pltpu.VMEM((C, D), jnp.float32),
          pltpu.SemaphoreType.DMA,
          pltpu.SemaphoreType.DMA,
          pltpu.SemaphoreType.DMA,
          pltpu.SemaphoreType.DMA,
          pltpu.SemaphoreType.DMA,
          pltpu.SemaphoreType.DMA,
          pltpu.SemaphoreType.DMA,
      ],
  )
  def sc_kernel(ids_hbm, table_hbm, nids_hbm, nw_hbm, xs_out, wn_out,
                idx_c, nids2, nflat, nw2, xs_v,
                rows_0, rows_1, rows_2, rows_3, out_v,
                sem0, sem1, sem2, sem_r0, sem_r1, sem_r2, sem_r3):
    wid = lax.axis_index("s") * NC + lax.axis_index("c")
    base = wid * ITEMS

    def chunk_body(g, carry):
      cbase = base + g * C
      pltpu.sync_copy(ids_hbm.at[pl.ds(cbase, C)], idx_c)
      cp0 = pltpu.async_copy(nids_hbm.at[idx_c], nids2, sem0)
      cp1 = pltpu.async_copy(nw_hbm.at[idx_c], nw2, sem1)
      cp2 = pltpu.async_copy(table_hbm.at[idx_c], xs_v, sem2)
      cp0.wait()
      cp1.wait()
      cp2.wait()

      def repack(i, c):
        nflat[pl.ds(i * K, K)] = nids2[i, :]
        return c

      lax.fori_loop(0, C, repack, 0, unroll=8)

      def fire(s, rows_buf, sem):
        return pltpu.async_copy(
            table_hbm.at[nflat.at[pl.ds(s * (SUB * K), SUB * K)]],
            rows_buf, sem)

      def wait_for(rows_buf, sem):
        pltpu.make_async_copy(
            table_hbm.at[nflat.at[pl.ds(0, SUB * K)]], rows_buf, sem).wait()

      def compute(s, rows_buf, c):
        def item_body(i, c2):
          accs = [jnp.zeros((LANES,), jnp.float32) for _ in range(DV)]
          wrow = nw2[s * SUB + i, :]
          for k in range(K):
            w = wrow[k]
            r = i * K + k
            for d in range(DV):
              accs[d] = accs[d] + w * rows_buf[r, pl.ds(d * LANES, LANES)]
          for d in range(DV):
            out_v[s * SUB + i, pl.ds(d * LANES, LANES)] = accs[d]
          return c2

        return lax.fori_loop(0, SUB, item_body, c)

      ring = [(rows_0, sem_r0), (rows_1, sem_r1), (rows_2, sem_r2),
              (rows_3, sem_r3)]
      for b in range(4):
        fire(b, *ring[b])

      def quad_body(qt, c):
        for b in range(4):
          s = qt * 4 + b
          buf, sem = ring[b]
          wait_for(buf, sem)
          c = compute(s, buf, c)

          @pl.when(s + 4 < NSUB)
          def _():
            fire(s + 4, buf, sem)
        return c

      lax.fori_loop(0, NSUB // 4, quad_body, 0)

      pltpu.sync_copy(xs_v, xs_out.at[pl.ds(cbase, C), :])
      pltpu.sync_copy(out_v, wn_out.at[pl.ds(cbase, C), :])
      return carry

    lax.fori_loop(0, NCHUNK, chunk_body, 0)

  return sc_kernel(node_ids, table, nbr_ids, nbr_w)


def _tc_linear_relu(xs, wn, Wn_w, Ws_w, Wc_w, bn, bs, b2):
  BM = 2048
  grid = (BSEG // BM,)

  def body(xs_ref, wn_ref, wnw_ref, wsw_ref, wcw_ref, bn_ref, bs_ref, b2_ref,
           o_ref):
    dn = (((1,), (1,)), ((), ()))
    h = lax.dot_general(wn_ref[...], wnw_ref[...], dn,
                        preferred_element_type=jnp.float32)
    h = h + lax.dot_general(xs_ref[...], wsw_ref[...], dn,
                            preferred_element_type=jnp.float32)
    h = h + bn_ref[...] + bs_ref[...]
    o = lax.dot_general(h, wcw_ref[...], dn,
                        preferred_element_type=jnp.float32)
    o_ref[...] = jnp.maximum(o + b2_ref[...], 0.0)

  return pl.pallas_call(
      body,
      grid=grid,
      in_specs=[
          pl.BlockSpec((BM, D), lambda i: (i, 0)),
          pl.BlockSpec((BM, D), lambda i: (i, 0)),
          pl.BlockSpec((D, D), lambda i: (0, 0)),
          pl.BlockSpec((D, D), lambda i: (0, 0)),
          pl.BlockSpec((D, D), lambda i: (0, 0)),
          pl.BlockSpec((1, D), lambda i: (0, 0)),
          pl.BlockSpec((1, D), lambda i: (0, 0)),
          pl.BlockSpec((1, D), lambda i: (0, 0)),
      ],
      out_specs=pl.BlockSpec((BM, D), lambda i: (i, 0)),
      out_shape=jax.ShapeDtypeStruct((BSEG, D), jnp.float32),
  )(xs, wn, Wn_w, Ws_w, Wc_w, bn, bs, b2)


def kernel(node_ids, global_emb_table, offline_nbr_ids, offline_nbr_weights,
           Wn_w, Wn_b, Ws_w, Ws_b, Wc_w, Wc_b):
  node_ids = node_ids.astype(jnp.int32)
  bn = Wn_b.reshape(1, D)
  bs = Ws_b.reshape(1, D)
  b2 = Wc_b.reshape(1, D)
  outs = []
  for seg in range(NSEG):
    ids_seg = lax.dynamic_slice_in_dim(node_ids, seg * BSEG, BSEG)
    xs, wnb = _sc_gather_weighted(ids_seg, global_emb_table,
                                  offline_nbr_ids, offline_nbr_weights)
    outs.append(_tc_linear_relu(xs, wnb, Wn_w, Ws_w, Wc_w, bn, bs, b2))
  return jnp.concatenate(outs, axis=0)


# speedup vs baseline: 1.0220x; 1.0220x over previous
import functools

import jax
import jax.numpy as jnp
from jax import lax
from jax.experimental import pallas as pl
from jax.experimental.pallas import tpu as pltpu
from jax.experimental.pallas import tpu_sc as plsc

B = 32768
K = 16
D = 128
NSEG = 2
BSEG = B // NSEG
NC = 2
NS = 16
NW = NC * NS
ITEMS = BSEG // NW
C = 64
NCHUNK = ITEMS // C
SUB = 8
NSUB = C // SUB
LANES = 16
DV = D // LANES
XITEMS = B // NW


def _sc_gather_xs(node_ids, table):
  mesh = plsc.VectorSubcoreMesh(core_axis_name="c", subcore_axis_name="s")

  @functools.partial(
      pl.kernel,
      out_type=jax.ShapeDtypeStruct((B, D), jnp.float32),
      mesh=mesh,
      compiler_params=pltpu.CompilerParams(use_tc_tiling_on_sc=False),
      scratch_types=[
          pltpu.VMEM((C,), jnp.int32),
          pltpu.VMEM((C, D), jnp.float32),
          pltpu.SemaphoreType.DMA,
      ],
  )
  def xs_kernel(ids_hbm, table_hbm, xs_out, idx_c, buf, sem):
    wid = lax.axis_index("s") * NC + lax.axis_index("c")
    base = wid * XITEMS

    def chunk_body(g, carry):
      cbase = base + g * C
      pltpu.sync_copy(ids_hbm.at[pl.ds(cbase, C)], idx_c)
      cp = pltpu.async_copy(table_hbm.at[idx_c], buf, sem)
      cp.wait()
      pltpu.sync_copy(buf, xs_out.at[pl.ds(cbase, C), :])
      return carry

    lax.fori_loop(0, XITEMS // C, chunk_body, 0)

  return xs_kernel(node_ids, table)


def _sc_gather_weighted(node_ids, table, nbr_ids, nbr_w):
  mesh = plsc.VectorSubcoreMesh(core_axis_name="c", subcore_axis_name="s")

  @functools.partial(
      pl.kernel,
      out_type=jax.ShapeDtypeStruct((BSEG, D), jnp.float32),
      mesh=mesh,
      compiler_params=pltpu.CompilerParams(use_tc_tiling_on_sc=False),
      scratch_types=[
          pltpu.VMEM((C,), jnp.int32),
          pltpu.VMEM((C, K), jnp.int32),
          pltpu.VMEM((C * K,), jnp.int32),
          pltpu.VMEM((C, K), jnp.float32),
          pltpu.VMEM((SUB * K, D), jnp.float32),
          pltpu.VMEM((SUB * K, D), jnp.float32),
          pltpu.VMEM((SUB * K, D), jnp.float32),
          pltpu.VMEM((SUB * K, D), jnp.float32),
          pltpu.VMEM((C, D), jnp.float32),
          pltpu.SemaphoreType.DMA,
          pltpu.SemaphoreType.DMA,
          pltpu.SemaphoreType.DMA,
          pltpu.SemaphoreType.DMA,
          pltpu.SemaphoreType.DMA,
          pltpu.SemaphoreType.DMA,
      ],
  )
  def sc_kernel(ids_hbm, table_hbm, nids_hbm, nw_hbm, wn_out,
                idx_c, nids2, nflat, nw2,
                rows_0, rows_1, rows_2, rows_3, out_v,
                sem0, sem1, sem_r0, sem_r1, sem_r2, sem_r3):
    wid = lax.axis_index("s") * NC + lax.axis_index("c")
    base = wid * ITEMS

    def chunk_body(g, carry):
      cbase = base + g * C
      pltpu.sync_copy(ids_hbm.at[pl.ds(cbase, C)], idx_c)
      cp0 = pltpu.async_copy(nids_hbm.at[idx_c], nids2, sem0)
      cp1 = pltpu.async_copy(nw_hbm.at[idx_c], nw2, sem1)
      cp0.wait()
      cp1.wait()

      def repack(i, c):
        nflat[pl.ds(i * K, K)] = nids2[i, :]
        return c

      lax.fori_loop(0, C, repack, 0, unroll=8)

      def fire(s, rows_buf, sem):
        return pltpu.async_copy(
            table_hbm.at[nflat.at[pl.ds(s * (SUB * K), SUB * K)]],
            rows_buf, sem)

      def wait_for(rows_buf, sem):
        pltpu.make_async_copy(
            table_hbm.at[nflat.at[pl.ds(0, SUB * K)]], rows_buf, sem).wait()

      def compute(s, rows_buf, c):
        def item_body(i, c2):
          accs = [jnp.zeros((LANES,), jnp.float32) for _ in range(DV)]
          wrow = nw2[s * SUB + i, :]
          for k in range(K):
            w = wrow[k]
            r = i * K + k
            for d in range(DV):
              accs[d] = accs[d] + w * rows_buf[r, pl.ds(d * LANES, LANES)]
          for d in range(DV):
            out_v[s * SUB + i, pl.ds(d * LANES, LANES)] = accs[d]
          return c2

        return lax.fori_loop(0, SUB, item_body, c)

      ring = [(rows_0, sem_r0), (rows_1, sem_r1), (rows_2, sem_r2),
              (rows_3, sem_r3)]
      for b in range(4):
        fire(b, *ring[b])

      def quad_body(qt, c):
        for b in range(4):
          s = qt * 4 + b
          buf, sem = ring[b]
          wait_for(buf, sem)
          c = compute(s, buf, c)

          @pl.when(s + 4 < NSUB)
          def _():
            fire(s + 4, buf, sem)
        return c

      lax.fori_loop(0, NSUB // 4, quad_body, 0)

      pltpu.sync_copy(out_v, wn_out.at[pl.ds(cbase, C), :])
      return carry

    lax.fori_loop(0, NCHUNK, chunk_body, 0)

  return sc_kernel(node_ids, table, nbr_ids, nbr_w)


def _tc_linear_relu(xs_full, wn, seg, Wn_w, Ws_w, Wc_w, bn, bs, b2):
  BM = 2048
  grid = (BSEG // BM,)
  off = seg * (BSEG // BM)

  def body(xs_ref, wn_ref, wnw_ref, wsw_ref, wcw_ref, bn_ref, bs_ref, b2_ref,
           o_ref):
    dn = (((1,), (1,)), ((), ()))
    h = lax.dot_general(wn_ref[...], wnw_ref[...], dn,
                        preferred_element_type=jnp.float32)
    h = h + lax.dot_general(xs_ref[...], wsw_ref[...], dn,
                            preferred_element_type=jnp.float32)
    h = h + bn_ref[...] + bs_ref[...]
    o = lax.dot_general(h, wcw_ref[...], dn,
                        preferred_element_type=jnp.float32)
    o_ref[...] = jnp.maximum(o + b2_ref[...], 0.0)

  return pl.pallas_call(
      body,
      grid=grid,
      in_specs=[
          pl.BlockSpec((BM, D), lambda i: (i + off, 0)),
          pl.BlockSpec((BM, D), lambda i: (i, 0)),
          pl.BlockSpec((D, D), lambda i: (0, 0)),
          pl.BlockSpec((D, D), lambda i: (0, 0)),
          pl.BlockSpec((D, D), lambda i: (0, 0)),
          pl.BlockSpec((1, D), lambda i: (0, 0)),
          pl.BlockSpec((1, D), lambda i: (0, 0)),
          pl.BlockSpec((1, D), lambda i: (0, 0)),
      ],
      out_specs=pl.BlockSpec((BM, D), lambda i: (i, 0)),
      out_shape=jax.ShapeDtypeStruct((BSEG, D), jnp.float32),
  )(xs_full, wn, Wn_w, Ws_w, Wc_w, bn, bs, b2)


def kernel(node_ids, global_emb_table, offline_nbr_ids, offline_nbr_weights,
           Wn_w, Wn_b, Ws_w, Ws_b, Wc_w, Wc_b):
  node_ids = node_ids.astype(jnp.int32)
  bn = Wn_b.reshape(1, D)
  bs = Ws_b.reshape(1, D)
  b2 = Wc_b.reshape(1, D)
  xs_full = _sc_gather_xs(node_ids, global_emb_table)
  outs = []
  for seg in range(NSEG):
    ids_seg = lax.dynamic_slice_in_dim(node_ids, seg * BSEG, BSEG)
    wnb = _sc_gather_weighted(ids_seg, global_emb_table,
                              offline_nbr_ids, offline_nbr_weights)
    outs.append(_tc_linear_relu(xs_full, wnb, seg, Wn_w, Ws_w, Wc_w,
                                bn, bs, b2))
  return jnp.concatenate(outs, axis=0)


# speedup vs baseline: 1.0481x; 1.0255x over previous
import functools

import jax
import jax.numpy as jnp
from jax import lax
from jax.experimental import pallas as pl
from jax.experimental.pallas import tpu as pltpu
from jax.experimental.pallas import tpu_sc as plsc

B = 32768
K = 16
D = 128
NSEG = 2
BSEG = B // NSEG
NC = 2
NS = 16
NW = NC * NS
ITEMS = BSEG // NW
C = 64
NCHUNK = ITEMS // C
SUB = 8
NSUB = C // SUB
LANES = 16
DV = D // LANES
XITEMS = B // NW


def _sc_gather_xs(node_ids, table):
  mesh = plsc.VectorSubcoreMesh(core_axis_name="c", subcore_axis_name="s")

  @functools.partial(
      pl.kernel,
      out_type=jax.ShapeDtypeStruct((B, D), jnp.float32),
      mesh=mesh,
      compiler_params=pltpu.CompilerParams(use_tc_tiling_on_sc=False),
      scratch_types=[
          pltpu.VMEM((C,), jnp.int32),
          pltpu.VMEM((C, D), jnp.float32),
          pltpu.SemaphoreType.DMA,
      ],
  )
  def xs_kernel(ids_hbm, table_hbm, xs_out, idx_c, buf, sem):
    wid = lax.axis_index("s") * NC + lax.axis_index("c")
    base = wid * XITEMS

    def chunk_body(g, carry):
      cbase = base + g * C
      pltpu.sync_copy(ids_hbm.at[pl.ds(cbase, C)], idx_c)
      cp = pltpu.async_copy(table_hbm.at[idx_c], buf, sem)
      cp.wait()
      pltpu.sync_copy(buf, xs_out.at[pl.ds(cbase, C), :])
      return carry

    lax.fori_loop(0, XITEMS // C, chunk_body, 0)

  return xs_kernel(node_ids, table)


def _sc_gather_weighted(node_ids, table, nbr_ids, nbr_w):
  mesh = plsc.VectorSubcoreMesh(core_axis_name="c", subcore_axis_name="s")

  @functools.partial(
      pl.kernel,
      out_type=jax.ShapeDtypeStruct((BSEG, D), jnp.float32),
      mesh=mesh,
      compiler_params=pltpu.CompilerParams(use_tc_tiling_on_sc=False),
      scratch_types=[
          pltpu.VMEM((C,), jnp.int32),
          pltpu.VMEM((C, K), jnp.int32),
          pltpu.VMEM((C * K,), jnp.int32),
          pltpu.VMEM((C, K), jnp.float32),
          pltpu.VMEM((SUB * K, D), jnp.float32),
          pltpu.VMEM((SUB * K, D), jnp.float32),
          pltpu.VMEM((SUB * K, D), jnp.float32),
          pltpu.VMEM((SUB * K, D), jnp.float32),
          pltpu.VMEM((C, D), jnp.float32),
          pltpu.SemaphoreType.DMA,
          pltpu.SemaphoreType.DMA,
          pltpu.SemaphoreType.DMA,
          pltpu.SemaphoreType.DMA,
          pltpu.SemaphoreType.DMA,
          pltpu.SemaphoreType.DMA,
      ],
  )
  def sc_kernel(ids_hbm, table_hbm, nids_hbm, nw_hbm, wn_out,
                idx_c, nids2, nflat, nw2,
                rows_0, rows_1, rows_2, rows_3, out_v,
                sem0, sem1, sem_r0, sem_r1, sem_r2, sem_r3):
    wid = lax.axis_index("s") * NC + lax.axis_index("c")
    base = wid * ITEMS

    def chunk_body(g, carry):
      cbase = base + g * C
      pltpu.sync_copy(ids_hbm.at[pl.ds(cbase, C)], idx_c)
      cp0 = pltpu.async_copy(nids_hbm.at[idx_c], nids2, sem0)
      cp1 = pltpu.async_copy(nw_hbm.at[idx_c], nw2, sem1)
      cp0.wait()
      cp1.wait()

      def repack(i, c):
        nflat[pl.ds(i * K, K)] = nids2[i, :]
        return c

      lax.fori_loop(0, C, repack, 0, unroll=8)

      def fire(s, rows_buf, sem):
        return pltpu.async_copy(
            table_hbm.at[nflat.at[pl.ds(s * (SUB * K), SUB * K)]],
            rows_buf, sem)

      def wait_for(rows_buf, sem):
        pltpu.make_async_copy(
            table_hbm.at[nflat.at[pl.ds(0, SUB * K)]], rows_buf, sem).wait()

      def compute(s, rows_buf, c):
        def item_body(i, c2):
          accs = [jnp.zeros((LANES,), jnp.float32) for _ in range(DV)]
          wrow = nw2[s * SUB + i, :]
          for k in range(K):
            w = wrow[k]
            r = i * K + k
            for d in range(DV):
              accs[d] = accs[d] + w * rows_buf[r, pl.ds(d * LANES, LANES)]
          for d in range(DV):
            out_v[s * SUB + i, pl.ds(d * LANES, LANES)] = accs[d]
          return c2

        return lax.fori_loop(0, SUB, item_body, c)

      ring = [(rows_0, sem_r0), (rows_1, sem_r1), (rows_2, sem_r2),
              (rows_3, sem_r3)]
      for b in range(4):
        fire(b, *ring[b])

      def quad_body(qt, c):
        for b in range(4):
          s = qt * 4 + b
          buf, sem = ring[b]
          wait_for(buf, sem)
          c = compute(s, buf, c)

          @pl.when(s + 4 < NSUB)
          def _():
            fire(s + 4, buf, sem)
        return c

      lax.fori_loop(0, NSUB // 4, quad_body, 0)

      pltpu.sync_copy(out_v, wn_out.at[pl.ds(cbase, C), :])
      return carry

    lax.fori_loop(0, NCHUNK, chunk_body, 0)

  return sc_kernel(node_ids, table, nbr_ids, nbr_w)


def _tc_linear_relu(xs_full, wn, seg, Wn_w, Ws_w, Wc_w, bn, bs, b2, prev):
  BM = 2048
  grid = (BSEG // BM,)
  off = seg * (BSEG // BM)

  def body(xs_ref, wn_ref, wnw_ref, wsw_ref, wcw_ref, bn_ref, bs_ref, b2_ref,
           prev_ref, o_ref):
    del prev_ref
    dn = (((1,), (1,)), ((), ()))
    h = lax.dot_general(wn_ref[...], wnw_ref[...], dn,
                        preferred_element_type=jnp.float32)
    h = h + lax.dot_general(xs_ref[...], wsw_ref[...], dn,
                            preferred_element_type=jnp.float32)
    h = h + bn_ref[...] + bs_ref[...]
    o = lax.dot_general(h, wcw_ref[...], dn,
                        preferred_element_type=jnp.float32)
    o_ref[...] = jnp.maximum(o + b2_ref[...], 0.0)

  in_specs = [
      pl.BlockSpec((BM, D), lambda i: (i + off, 0)),
      pl.BlockSpec((BM, D), lambda i: (i, 0)),
      pl.BlockSpec((D, D), lambda i: (0, 0)),
      pl.BlockSpec((D, D), lambda i: (0, 0)),
      pl.BlockSpec((D, D), lambda i: (0, 0)),
      pl.BlockSpec((1, D), lambda i: (0, 0)),
      pl.BlockSpec((1, D), lambda i: (0, 0)),
      pl.BlockSpec((1, D), lambda i: (0, 0)),
      pl.BlockSpec((BM, D), lambda i: (i + off, 0)),
  ]
  return pl.pallas_call(
      body,
      grid=grid,
      in_specs=in_specs,
      out_specs=pl.BlockSpec((BM, D), lambda i: (i + off, 0)),
      out_shape=jax.ShapeDtypeStruct((B, D), jnp.float32),
      input_output_aliases={8: 0},
  )(xs_full, wn, Wn_w, Ws_w, Wc_w, bn, bs, b2, prev)


def kernel(node_ids, global_emb_table, offline_nbr_ids, offline_nbr_weights,
           Wn_w, Wn_b, Ws_w, Ws_b, Wc_w, Wc_b):
  node_ids = node_ids.astype(jnp.int32)
  bn = Wn_b.reshape(1, D)
  bs = Ws_b.reshape(1, D)
  b2 = Wc_b.reshape(1, D)
  xs_full = _sc_gather_xs(node_ids, global_emb_table)
  out = jnp.zeros((B, D), jnp.float32)
  for seg in range(NSEG):
    ids_seg = lax.dynamic_slice_in_dim(node_ids, seg * BSEG, BSEG)
    wnb = _sc_gather_weighted(ids_seg, global_emb_table,
                              offline_nbr_ids, offline_nbr_weights)
    out = _tc_linear_relu(xs_full, wnb, seg, Wn_w, Ws_w, Wc_w,
                          bn, bs, b2, out)
  return out


# speedup vs baseline: 1.0602x; 1.0116x over previous
import functools

import jax
import jax.numpy as jnp
from jax import lax
from jax.experimental import pallas as pl
from jax.experimental.pallas import tpu as pltpu
from jax.experimental.pallas import tpu_sc as plsc

B = 32768
K = 16
D = 128
NSEG = 2
BSEG = B // NSEG
NC = 2
NS = 16
NW = NC * NS
ITEMS = BSEG // NW
C = 64
NCHUNK = ITEMS // C
SUB = 8
NSUB = C // SUB
LANES = 16
DV = D // LANES
XITEMS = B // NW


def _sc_gather_xs(node_ids, table):
  mesh = plsc.VectorSubcoreMesh(core_axis_name="c", subcore_axis_name="s")

  @functools.partial(
      pl.kernel,
      out_type=jax.ShapeDtypeStruct((B, D), jnp.float32),
      mesh=mesh,
      compiler_params=pltpu.CompilerParams(use_tc_tiling_on_sc=False),
      scratch_types=[
          pltpu.VMEM((C,), jnp.int32),
          pltpu.VMEM((C, D), jnp.float32),
          pltpu.SemaphoreType.DMA,
      ],
  )
  def xs_kernel(ids_hbm, table_hbm, xs_out, idx_c, buf, sem):
    wid = lax.axis_index("s") * NC + lax.axis_index("c")
    base = wid * XITEMS

    def chunk_body(g, carry):
      cbase = base + g * C
      pltpu.sync_copy(ids_hbm.at[pl.ds(cbase, C)], idx_c)
      cp = pltpu.async_copy(table_hbm.at[idx_c], buf, sem)
      cp.wait()
      pltpu.sync_copy(buf, xs_out.at[pl.ds(cbase, C), :])
      return carry

    lax.fori_loop(0, XITEMS // C, chunk_body, 0)

  return xs_kernel(node_ids, table)


def _sc_gather_weighted(node_ids, table, nbr_ids, nbr_w):
  mesh = plsc.VectorSubcoreMesh(core_axis_name="c", subcore_axis_name="s")

  @functools.partial(
      pl.kernel,
      out_type=jax.ShapeDtypeStruct((BSEG, D), jnp.float32),
      mesh=mesh,
      compiler_params=pltpu.CompilerParams(use_tc_tiling_on_sc=False),
      scratch_types=[
          pltpu.VMEM((C,), jnp.int32),
          pltpu.VMEM((C, K), jnp.int32),
          pltpu.VMEM((C * K,), jnp.int32),
          pltpu.VMEM((C, K), jnp.float32),
          pltpu.VMEM((SUB * K, D), jnp.float32),
          pltpu.VMEM((SUB * K, D), jnp.float32),
          pltpu.VMEM((SUB * K, D), jnp.float32),
          pltpu.VMEM((SUB * K, D), jnp.float32),
          pltpu.VMEM((C, D), jnp.float32),
          pltpu.SemaphoreType.DMA,
          pltpu.SemaphoreType.DMA,
          pltpu.SemaphoreType.DMA,
          pltpu.SemaphoreType.DMA,
          pltpu.SemaphoreType.DMA,
          pltpu.SemaphoreType.DMA,
      ],
  )
  def sc_kernel(ids_hbm, table_hbm, nids_hbm, nw_hbm, wn_out,
                idx_c, nids2, nflat, nw2,
                rows_0, rows_1, rows_2, rows_3, out_v,
                sem0, sem1, sem_r0, sem_r1, sem_r2, sem_r3):
    wid = lax.axis_index("s") * NC + lax.axis_index("c")
    base = wid * ITEMS

    def chunk_body(g, carry):
      cbase = base + g * C
      pltpu.sync_copy(ids_hbm.at[pl.ds(cbase, C)], idx_c)
      cp0 = pltpu.async_copy(nids_hbm.at[idx_c], nids2, sem0)
      cp1 = pltpu.async_copy(nw_hbm.at[idx_c], nw2, sem1)
      cp0.wait()
      cp1.wait()

      def repack(i, c):
        nflat[pl.ds(i * K, K)] = nids2[i, :]
        return c

      lax.fori_loop(0, C, repack, 0, unroll=8)

      def fire(s, rows_buf, sem):
        return pltpu.async_copy(
            table_hbm.at[nflat.at[pl.ds(s * (SUB * K), SUB * K)]],
            rows_buf, sem)

      def wait_for(rows_buf, sem):
        pltpu.make_async_copy(
            table_hbm.at[nflat.at[pl.ds(0, SUB * K)]], rows_buf, sem).wait()

      def compute(s, rows_buf, c):
        def item_body(i, c2):
          accs = [jnp.zeros((LANES,), jnp.float32) for _ in range(DV)]
          wrow = nw2[s * SUB + i, :]
          for k in range(K):
            w = wrow[k]
            r = i * K + k
            for d in range(DV):
              accs[d] = accs[d] + w * rows_buf[r, pl.ds(d * LANES, LANES)]
          for d in range(DV):
            out_v[s * SUB + i, pl.ds(d * LANES, LANES)] = accs[d]
          return c2

        return lax.fori_loop(0, SUB, item_body, c)

      ring = [(rows_0, sem_r0), (rows_1, sem_r1), (rows_2, sem_r2),
              (rows_3, sem_r3)]
      for b in range(4):
        fire(b, *ring[b])

      def quad_body(qt, c):
        for b in range(4):
          s = qt * 4 + b
          buf, sem = ring[b]
          wait_for(buf, sem)
          c = compute(s, buf, c)

          @pl.when(s + 4 < NSUB)
          def _():
            fire(s + 4, buf, sem)
        return c

      lax.fori_loop(0, NSUB // 4, quad_body, 0)

      pltpu.sync_copy(out_v, wn_out.at[pl.ds(cbase, C), :])
      return carry

    lax.fori_loop(0, NCHUNK, chunk_body, 0)

  return sc_kernel(node_ids, table, nbr_ids, nbr_w)


def _tc_linear_relu(xs_full, wn, seg, Wn_w, Ws_w, Wc_w, bn, bs, b2, prev):
  BM = 2048
  grid = (BSEG // BM,)
  off = seg * (BSEG // BM)

  def body(xs_ref, wn_ref, wnw_ref, wsw_ref, wcw_ref, bn_ref, bs_ref, b2_ref,
           *prev_and_out):
    o_ref = prev_and_out[-1]
    dn = (((1,), (1,)), ((), ()))
    h = lax.dot_general(wn_ref[...], wnw_ref[...], dn,
                        preferred_element_type=jnp.float32)
    h = h + lax.dot_general(xs_ref[...], wsw_ref[...], dn,
                            preferred_element_type=jnp.float32)
    h = h + bn_ref[...] + bs_ref[...]
    o = lax.dot_general(h, wcw_ref[...], dn,
                        preferred_element_type=jnp.float32)
    o_ref[...] = jnp.maximum(o + b2_ref[...], 0.0)

  in_specs = [
      pl.BlockSpec((BM, D), lambda i: (i + off, 0)),
      pl.BlockSpec((BM, D), lambda i: (i, 0)),
      pl.BlockSpec((D, D), lambda i: (0, 0)),
      pl.BlockSpec((D, D), lambda i: (0, 0)),
      pl.BlockSpec((D, D), lambda i: (0, 0)),
      pl.BlockSpec((1, D), lambda i: (0, 0)),
      pl.BlockSpec((1, D), lambda i: (0, 0)),
      pl.BlockSpec((1, D), lambda i: (0, 0)),
  ]
  args = [xs_full, wn, Wn_w, Ws_w, Wc_w, bn, bs, b2]
  aliases = {}
  if prev is not None:
    in_specs.append(pl.BlockSpec((BM, D), lambda i: (i + off, 0)))
    args.append(prev)
    aliases = {8: 0}
  return pl.pallas_call(
      body,
      grid=grid,
      in_specs=in_specs,
      out_specs=pl.BlockSpec((BM, D), lambda i: (i + off, 0)),
      out_shape=jax.ShapeDtypeStruct((B, D), jnp.float32),
      input_output_aliases=aliases,
  )(*args)


def kernel(node_ids, global_emb_table, offline_nbr_ids, offline_nbr_weights,
           Wn_w, Wn_b, Ws_w, Ws_b, Wc_w, Wc_b):
  node_ids = node_ids.astype(jnp.int32)
  bn = Wn_b.reshape(1, D)
  bs = Ws_b.reshape(1, D)
  b2 = Wc_b.reshape(1, D)
  xs_full = _sc_gather_xs(node_ids, global_emb_table)
  out = None
  for seg in range(NSEG):
    ids_seg = lax.dynamic_slice_in_dim(node_ids, seg * BSEG, BSEG)
    wnb = _sc_gather_weighted(ids_seg, global_emb_table,
                              offline_nbr_ids, offline_nbr_weights)
    out = _tc_linear_relu(xs_full, wnb, seg, Wn_w, Ws_w, Wc_w,
                          bn, bs, b2, out)
  return out
